# Initial kernel scaffold; baseline (speedup 1.0000x reference)
import jax
import jax.numpy as jnp
from jax.experimental import pallas as pl


def kernel(warped_sample_points, voxel_grid):
    raise NotImplementedError("write your pallas kernel here")


# speedup vs baseline: 2.1134x; 2.1134x over previous
import functools

import jax
import jax.numpy as jnp
from jax import lax
from jax.experimental import pallas as pl
from jax.experimental.pallas import tpu as pltpu
from jax.experimental.pallas import tpu_sc as plsc

B = 262144
C = 32
D = H = W = 128
DHW = D * H * W

NC = 2
NS = 16
NW = NC * NS
PW = B // NW
P = 128
NCHUNK = PW // P
L = 16


def _axis_coords(p):
    f = ((p + 1.0) * 0.5) * 127.0
    i0 = f.astype(jnp.int32)
    fr = f - i0.astype(jnp.float32)
    i1 = jnp.minimum(i0 + 1, 127)
    return i0, i1, fr


mesh = plsc.VectorSubcoreMesh(core_axis_name="c", subcore_axis_name="s")


@functools.partial(
    pl.kernel,
    out_type=jax.ShapeDtypeStruct((B, C), jnp.float32),
    mesh=mesh,
    scratch_types=[
        pltpu.VMEM((P,), jnp.float32),
        pltpu.VMEM((P,), jnp.float32),
        pltpu.VMEM((P,), jnp.float32),
        pltpu.VMEM((8, P), jnp.int32),
        pltpu.VMEM((8 * P,), jnp.float32),
        pltpu.VMEM((8 * P, C), jnp.float32),
        pltpu.VMEM((P, C), jnp.float32),
        pltpu.SemaphoreType.DMA,
    ],
    compiler_params=pltpu.CompilerParams(use_tc_tiling_on_sc=False),
)
def _sc_interp(pts_hbm, table_hbm, out_hbm,
               zv, yv, xv, idx_v, w8_v, rows_v, out_v, gsem):
    wid = lax.axis_index("s") * NC + lax.axis_index("c")
    base = wid * PW

    def chunk_body(g, carry):
        row0 = base + g * P
        pltpu.sync_copy(pts_hbm.at[pl.ds(row0, P)], zv)
        pltpu.sync_copy(pts_hbm.at[pl.ds(B + row0, P)], yv)
        pltpu.sync_copy(pts_hbm.at[pl.ds(2 * B + row0, P)], xv)

        for t in range(P // L):
            s = t * L
            sl = pl.ds(s, L)
            zi0, zi1, fz = _axis_coords(zv[sl])
            yi0, yi1, fy = _axis_coords(yv[sl])
            xi0, xi1, fx = _axis_coords(xv[sl])
            zy00 = zi0 * (H * W) + yi0 * W
            zy01 = zi0 * (H * W) + yi1 * W
            zy10 = zi1 * (H * W) + yi0 * W
            zy11 = zi1 * (H * W) + yi1 * W
            idx_v[0, sl] = zy00 + xi0
            idx_v[1, sl] = zy00 + xi1
            idx_v[2, sl] = zy01 + xi0
            idx_v[3, sl] = zy01 + xi1
            idx_v[4, sl] = zy10 + xi0
            idx_v[5, sl] = zy10 + xi1
            idx_v[6, sl] = zy11 + xi0
            idx_v[7, sl] = zy11 + xi1
            fz0 = 1.0 - fz
            fy0 = 1.0 - fy
            fx0 = 1.0 - fx
            m00 = fz0 * fy0
            m01 = fz0 * fy
            m10 = fz * fy0
            m11 = fz * fy
            w8_v[pl.ds(0 * P + s, L)] = m00 * fx0
            w8_v[pl.ds(1 * P + s, L)] = m00 * fx
            w8_v[pl.ds(2 * P + s, L)] = m01 * fx0
            w8_v[pl.ds(3 * P + s, L)] = m01 * fx
            w8_v[pl.ds(4 * P + s, L)] = m10 * fx0
            w8_v[pl.ds(5 * P + s, L)] = m10 * fx
            w8_v[pl.ds(6 * P + s, L)] = m11 * fx0
            w8_v[pl.ds(7 * P + s, L)] = m11 * fx

        copies = [
            pltpu.async_copy(table_hbm.at[idx_v.at[k]],
                             rows_v.at[pl.ds(k * P, P)], gsem)
            for k in range(8)
        ]
        for cp in copies:
            cp.wait()

        def grp_body(t, carry2):
            jbase = t * L
            wrows = [w8_v[pl.ds(k * P + jbase, L)] for k in range(8)]
            for jj in range(L):
                j = jbase + jj
                lane = jnp.full((L,), jj, jnp.int32)
                acc0 = jnp.zeros((L,), jnp.float32)
                acc1 = jnp.zeros((L,), jnp.float32)
                for k in range(8):
                    wb = wrows[k][lane]
                    acc0 = acc0 + wb * rows_v[k * P + j, pl.ds(0, L)]
                    acc1 = acc1 + wb * rows_v[k * P + j, pl.ds(L, L)]
                out_v[j, pl.ds(0, L)] = acc0
                out_v[j, pl.ds(L, L)] = acc1
            return carry2

        lax.fori_loop(0, P // L, grp_body, 0)
        pltpu.sync_copy(out_v, out_hbm.at[pl.ds(row0, P)])
        return carry

    lax.fori_loop(0, NCHUNK, chunk_body, 0)


def kernel(warped_sample_points, voxel_grid):
    table = voxel_grid[0].transpose(1, 2, 3, 0).reshape(DHW, C)
    pts = warped_sample_points.T.reshape(3 * B)
    return _sc_interp(pts, table)


# speedup vs baseline: 2.1147x; 1.0006x over previous
import functools

import jax
import jax.numpy as jnp
from jax import lax
from jax.experimental import pallas as pl
from jax.experimental.pallas import tpu as pltpu
from jax.experimental.pallas import tpu_sc as plsc

B = 262144
C = 32
D = H = W = 128
DHW = D * H * W

NC = 2
NS = 16
NW = NC * NS
PW = B // NW
P = 128
NCHUNK = PW // P
L = 16


def _axis_coords(p):
    f = ((p + 1.0) * 0.5) * 127.0
    i0 = f.astype(jnp.int32)
    fr = f - i0.astype(jnp.float32)
    i1 = jnp.minimum(i0 + 1, 127)
    return i0, i1, fr


mesh = plsc.VectorSubcoreMesh(core_axis_name="c", subcore_axis_name="s")


@functools.partial(
    pl.kernel,
    out_type=jax.ShapeDtypeStruct((B, C), jnp.float32),
    mesh=mesh,
    scratch_types=[
        pltpu.VMEM((P,), jnp.float32),
        pltpu.VMEM((P,), jnp.float32),
        pltpu.VMEM((P,), jnp.float32),
        pltpu.VMEM((8, P), jnp.int32),
        pltpu.VMEM((8 * P,), jnp.float32),
        pltpu.VMEM((8 * P, C), jnp.float32),
        pltpu.VMEM((P, C), jnp.float32),
        pltpu.SemaphoreType.DMA,
    ],
    compiler_params=pltpu.CompilerParams(use_tc_tiling_on_sc=False),
)
def _sc_interp(pts_hbm, table_hbm, out_hbm,
               zv, yv, xv, idx_v, w8_v, rows_v, out_v, gsem):
    wid = lax.axis_index("s") * NC + lax.axis_index("c")
    base = wid * PW

    def chunk_body(g, carry):
        row0 = base + g * P
        pltpu.sync_copy(pts_hbm.at[pl.ds(row0, P)], zv)
        pltpu.sync_copy(pts_hbm.at[pl.ds(B + row0, P)], yv)
        pltpu.sync_copy(pts_hbm.at[pl.ds(2 * B + row0, P)], xv)

        for t in range(P // L):
            s = t * L
            sl = pl.ds(s, L)
            zi0, zi1, fz = _axis_coords(zv[sl])
            yi0, yi1, fy = _axis_coords(yv[sl])
            xi0, xi1, fx = _axis_coords(xv[sl])
            zy00 = zi0 * (H * W) + yi0 * W
            zy01 = zi0 * (H * W) + yi1 * W
            zy10 = zi1 * (H * W) + yi0 * W
            zy11 = zi1 * (H * W) + yi1 * W
            idx_v[0, sl] = zy00 + xi0
            idx_v[1, sl] = zy00 + xi1
            idx_v[2, sl] = zy01 + xi0
            idx_v[3, sl] = zy01 + xi1
            idx_v[4, sl] = zy10 + xi0
            idx_v[5, sl] = zy10 + xi1
            idx_v[6, sl] = zy11 + xi0
            idx_v[7, sl] = zy11 + xi1
            fz0 = 1.0 - fz
            fy0 = 1.0 - fy
            fx0 = 1.0 - fx
            m00 = fz0 * fy0
            m01 = fz0 * fy
            m10 = fz * fy0
            m11 = fz * fy
            w8_v[pl.ds(0 * P + s, L)] = m00 * fx0
            w8_v[pl.ds(1 * P + s, L)] = m00 * fx
            w8_v[pl.ds(2 * P + s, L)] = m01 * fx0
            w8_v[pl.ds(3 * P + s, L)] = m01 * fx
            w8_v[pl.ds(4 * P + s, L)] = m10 * fx0
            w8_v[pl.ds(5 * P + s, L)] = m10 * fx
            w8_v[pl.ds(6 * P + s, L)] = m11 * fx0
            w8_v[pl.ds(7 * P + s, L)] = m11 * fx

        copies = [
            pltpu.async_copy(table_hbm.at[idx_v.at[k]],
                             rows_v.at[pl.ds(k * P, P)], gsem)
            for k in range(8)
        ]
        for cp in copies:
            cp.wait()

        def grp_body(t, carry2):
            jbase = t * L
            wrows = [w8_v[pl.ds(k * P + jbase, L)] for k in range(8)]
            for jj in range(L):
                j = jbase + jj
                lane = jnp.full((L,), jj, jnp.int32)
                acc0 = jnp.zeros((L,), jnp.float32)
                acc1 = jnp.zeros((L,), jnp.float32)
                for k in range(8):
                    wb = wrows[k][lane]
                    acc0 = acc0 + wb * rows_v[k * P + j, pl.ds(0, L)]
                    acc1 = acc1 + wb * rows_v[k * P + j, pl.ds(L, L)]
                out_v[j, pl.ds(0, L)] = acc0
                out_v[j, pl.ds(L, L)] = acc1
            return carry2

        lax.fori_loop(0, P // L, grp_body, 0)
        pltpu.sync_copy(out_v, out_hbm.at[pl.ds(row0, P)])
        return carry

    lax.fori_loop(0, NCHUNK, chunk_body, 0)


def kernel(warped_sample_points, voxel_grid):
    table128 = voxel_grid[0].transpose(1, 2, 3, 0).reshape(DHW * C // 128, 128)
    table128 = lax.optimization_barrier(table128)
    table = table128.reshape(DHW, C)
    pts = warped_sample_points.T.reshape(3 * B)
    return _sc_interp(pts, table)
